# baseline (speedup 1.0000x reference)
import jax
import jax.numpy as jnp
from jax import lax
from jax.experimental import pallas as pl
from jax.experimental.pallas import tpu as pltpu
from jax.experimental.pallas import tpu_sc as plsc

EMB = 64
SEQ = 201
BATCH_ROWS = 1024
VOCAB_ROWS = 1000002
TBLK = 8192
NBLK = -(-VOCAB_ROWS // TBLK)
ROWS_LIN = NBLK * TBLK
NUM_CORES = 2
NUM_SUBCORES = 16
NW = NUM_CORES * NUM_SUBCORES
ROWS_PER_W = BATCH_ROWS // NW
SCALE = 8.0
C0 = 128
C1 = SEQ - C0
NBUF = 4
PF = 2


HALF = TBLK // 2
_LAST_BLK = VOCAB_ROWS // HALF
_LOG2_HALF = HALF.bit_length() - 1


def _linearize_body(a_ref, b_ref, o_ref):
    ya = jnp.transpose(a_ref[...], (1, 0))
    yb = jnp.transpose(b_ref[...], (1, 0))
    o_ref[...] = jnp.concatenate([ya, yb], axis=1) * SCALE


_linearize = pl.pallas_call(
    _linearize_body,
    grid=(NBLK,),
    in_specs=[
        pl.BlockSpec((EMB, HALF), lambda i: (0, jnp.minimum(2 * i, _LAST_BLK))),
        pl.BlockSpec((EMB, HALF), lambda i: (0, jnp.minimum(2 * i + 1, _LAST_BLK))),
    ],
    out_specs=pl.BlockSpec((HALF, 2 * EMB), lambda i: (i, 0)),
    out_shape=jax.ShapeDtypeStruct((NBLK * HALF, 2 * EMB), jnp.float32),
)


def _body(batch_hbm, table_hbm, pos_hbm, out_hbm, idx_all, pos_v,
          b0, b1, b2, b3, g0, g1, g2, g3, s0, s1, s2, s3):
    bufs = (b0, b1, b2, b3)
    gsems = (g0, g1, g2, g3)
    ssems = (s0, s1, s2, s3)
    wid = lax.axis_index("s") * NUM_CORES + lax.axis_index("c")
    base = wid * ROWS_PER_W
    pltpu.sync_copy(batch_hbm.at[pl.ds(base, ROWS_PER_W)], idx_all)
    pltpu.sync_copy(pos_hbm, pos_v)

    def fire_gather(j):
        k = j % NBUF
        cp0 = pltpu.async_copy(
            table_hbm.at[idx_all.at[j, pl.ds(0, C0)]],
            bufs[k].at[pl.ds(0, C0)], gsems[k],
        )
        cp1 = pltpu.async_copy(
            table_hbm.at[idx_all.at[j, pl.ds(C0, C1)]],
            bufs[k].at[pl.ds(C0, C1)], gsems[k],
        )
        return (cp0, cp1)

    gath = {}
    scat = {}
    for j in range(PF):
        gath[j] = fire_gather(j)

    for j in range(ROWS_PER_W):
        k = j % NBUF
        jn = j + PF
        if jn < ROWS_PER_W:
            kn = jn % NBUF
            if kn in scat:
                scat.pop(kn).wait()
            gath[jn] = fire_gather(jn)
        cp0, cp1 = gath.pop(j)
        cp0.wait()
        cp1.wait()

        buf = bufs[k]

        @pl.loop(0, SEQ, unroll=3)
        def _fma(r):
            for c in range(EMB // 16):
                seg = pl.ds(c * 16, 16)
                buf[r, seg] = buf[r, seg] + pos_v[r, seg]

        scat[k] = pltpu.async_copy(buf, out_hbm.at[base + j], ssems[k])

    for k in list(scat):
        scat.pop(k).wait()


_encodings = pl.kernel(
    _body,
    out_type=jax.ShapeDtypeStruct((BATCH_ROWS, SEQ, EMB), jnp.float32),
    mesh=plsc.VectorSubcoreMesh(core_axis_name="c", subcore_axis_name="s"),
    compiler_params=pltpu.CompilerParams(use_tc_tiling_on_sc=False),
    scratch_types=[
        pltpu.VMEM((ROWS_PER_W, SEQ), jnp.int32),
        pltpu.VMEM((SEQ, EMB), jnp.float32),
        pltpu.VMEM((SEQ, EMB), jnp.float32),
        pltpu.VMEM((SEQ, EMB), jnp.float32),
        pltpu.VMEM((SEQ, EMB), jnp.float32),
        pltpu.VMEM((SEQ, EMB), jnp.float32),
        pltpu.SemaphoreType.DMA,
        pltpu.SemaphoreType.DMA,
        pltpu.SemaphoreType.DMA,
        pltpu.SemaphoreType.DMA,
        pltpu.SemaphoreType.DMA,
        pltpu.SemaphoreType.DMA,
        pltpu.SemaphoreType.DMA,
        pltpu.SemaphoreType.DMA,
    ],
)


@jax.jit
def kernel(batch, table, pos_table):
    table_t = table.T
    lin = _linearize(table_t, table_t)
    table_lin = jnp.reshape(lin, (ROWS_LIN, EMB))
    t = batch.astype(jnp.int32)
    rt = (t & ~(TBLK - 1)) | ((t & (HALF - 1)) << 1) | ((t >> _LOG2_HALF) & 1)
    return _encodings(rt, table_lin, pos_table)


# speedup vs baseline: 1.0668x; 1.0668x over previous
import jax
import jax.numpy as jnp
from jax import lax
from jax.experimental import pallas as pl
from jax.experimental.pallas import tpu as pltpu
from jax.experimental.pallas import tpu_sc as plsc

EMB = 64
SEQ = 201
BATCH_ROWS = 1024
VOCAB_ROWS = 1000002
TBLK = 16384
NBLK = -(-VOCAB_ROWS // TBLK)
ROWS_LIN = NBLK * TBLK
NUM_CORES = 2
NUM_SUBCORES = 16
NW = NUM_CORES * NUM_SUBCORES
ROWS_PER_W = BATCH_ROWS // NW
SCALE = 8.0
C0 = 128
C1 = SEQ - C0
NBUF = 4
PF = 2


HALF = TBLK // 2
_LAST_BLK = VOCAB_ROWS // HALF
_LOG2_HALF = HALF.bit_length() - 1


def _linearize_body(a_ref, b_ref, o_ref):
    ya = jnp.transpose(a_ref[...], (1, 0))
    yb = jnp.transpose(b_ref[...], (1, 0))
    o_ref[...] = jnp.concatenate([ya, yb], axis=1) * SCALE


_linearize = pl.pallas_call(
    _linearize_body,
    grid=(NBLK,),
    in_specs=[
        pl.BlockSpec((EMB, HALF), lambda i: (0, jnp.minimum(2 * i, _LAST_BLK))),
        pl.BlockSpec((EMB, HALF), lambda i: (0, jnp.minimum(2 * i + 1, _LAST_BLK))),
    ],
    out_specs=pl.BlockSpec((HALF, 2 * EMB), lambda i: (i, 0)),
    out_shape=jax.ShapeDtypeStruct((NBLK * HALF, 2 * EMB), jnp.float32),
)


def _body(batch_hbm, table_hbm, pos_hbm, out_hbm, idx_all, pos_v,
          b0, b1, b2, b3, g0, g1, g2, g3, s0, s1, s2, s3):
    bufs = (b0, b1, b2, b3)
    gsems = (g0, g1, g2, g3)
    ssems = (s0, s1, s2, s3)
    wid = lax.axis_index("s") * NUM_CORES + lax.axis_index("c")
    base = wid * ROWS_PER_W
    pltpu.sync_copy(batch_hbm.at[pl.ds(base, ROWS_PER_W)], idx_all)
    pltpu.sync_copy(pos_hbm, pos_v)

    def fire_gather(j):
        k = j % NBUF
        cp0 = pltpu.async_copy(
            table_hbm.at[idx_all.at[j, pl.ds(0, C0)]],
            bufs[k].at[pl.ds(0, C0)], gsems[k],
        )
        cp1 = pltpu.async_copy(
            table_hbm.at[idx_all.at[j, pl.ds(C0, C1)]],
            bufs[k].at[pl.ds(C0, C1)], gsems[k],
        )
        return (cp0, cp1)

    gath = {}
    scat = {}
    for j in range(PF):
        gath[j] = fire_gather(j)

    for j in range(ROWS_PER_W):
        k = j % NBUF
        jn = j + PF
        if jn < ROWS_PER_W:
            kn = jn % NBUF
            if kn in scat:
                scat.pop(kn).wait()
            gath[jn] = fire_gather(jn)
        cp0, cp1 = gath.pop(j)
        cp0.wait()
        cp1.wait()

        buf = bufs[k]

        @pl.loop(0, SEQ, unroll=3)
        def _fma(r):
            for c in range(EMB // 16):
                seg = pl.ds(c * 16, 16)
                buf[r, seg] = buf[r, seg] + pos_v[r, seg]

        scat[k] = pltpu.async_copy(buf, out_hbm.at[base + j], ssems[k])

    for k in list(scat):
        scat.pop(k).wait()


_encodings = pl.kernel(
    _body,
    out_type=jax.ShapeDtypeStruct((BATCH_ROWS, SEQ, EMB), jnp.float32),
    mesh=plsc.VectorSubcoreMesh(core_axis_name="c", subcore_axis_name="s"),
    compiler_params=pltpu.CompilerParams(use_tc_tiling_on_sc=False),
    scratch_types=[
        pltpu.VMEM((ROWS_PER_W, SEQ), jnp.int32),
        pltpu.VMEM((SEQ, EMB), jnp.float32),
        pltpu.VMEM((SEQ, EMB), jnp.float32),
        pltpu.VMEM((SEQ, EMB), jnp.float32),
        pltpu.VMEM((SEQ, EMB), jnp.float32),
        pltpu.VMEM((SEQ, EMB), jnp.float32),
        pltpu.SemaphoreType.DMA,
        pltpu.SemaphoreType.DMA,
        pltpu.SemaphoreType.DMA,
        pltpu.SemaphoreType.DMA,
        pltpu.SemaphoreType.DMA,
        pltpu.SemaphoreType.DMA,
        pltpu.SemaphoreType.DMA,
        pltpu.SemaphoreType.DMA,
    ],
)


@jax.jit
def kernel(batch, table, pos_table):
    table_t = table.T
    lin = _linearize(table_t, table_t)
    table_lin = jnp.reshape(lin, (ROWS_LIN, EMB))
    t = batch.astype(jnp.int32)
    rt = (t & ~(TBLK - 1)) | ((t & (HALF - 1)) << 1) | ((t >> _LOG2_HALF) & 1)
    return _encodings(rt, table_lin, pos_table)
